# baseline (speedup 1.0000x reference)
import functools

import numpy as np
import jax
import jax.numpy as jnp
from jax import lax
from jax.experimental import pallas as pl
from jax.experimental.pallas import tpu as pltpu
from jax.experimental.pallas import tpu_sc as plsc

MIN_SCALE = np.float32(1e-06)
ALPHA = np.float32(0.5)
EPS = np.float32(1e-08)
OBS_SCALE = np.float32(0.5)
DT = np.float32(0.1)

B = 16
N = 4096
A = 64

_KEY = np.array([2247515013, 2545468385], dtype=np.uint32)
_KS = np.array(
    [_KEY[0], _KEY[1], _KEY[0] ^ _KEY[1] ^ np.uint32(0x1BD11BDA)],
    dtype=np.uint32,
).view(np.int32)
_ROTS = (13, 15, 26, 6, 17, 29, 16, 24)
_TINY = np.float32(np.finfo(np.float32).tiny)

_VAR = np.float32(max(OBS_SCALE * OBS_SCALE, MIN_SCALE))
_LOG_VAR = np.float32(np.log(_VAR))
_LOG_2PI = np.float32(np.log(np.float32(2.0 * np.pi)))

D_CHUNK = 8
ROWS = D_CHUNK * B
HALF = N // 2

S_SC = 1440
SC_ROWS = S_SC * B
ROWS_PW = SC_ROWS // 32
TR_PW = ROWS_PW // 8


def _rotl(x, r):
    return lax.shift_left(x, np.int32(r)) | lax.shift_right_logical(
        x, np.int32(32 - r))


def _threefry_bits(cnt):
    ks0, ks1, ks2 = (np.int32(_KS[0]), np.int32(_KS[1]), np.int32(_KS[2]))
    x0 = jnp.full(cnt.shape, ks0, jnp.int32)
    x1 = cnt + ks1

    def rounds(x0, x1, rots):
        for r in rots:
            x0 = x0 + x1
            x1 = _rotl(x1, r)
            x1 = x0 ^ x1
        return x0, x1

    x0, x1 = rounds(x0, x1, _ROTS[:4])
    x0 = x0 + ks1
    x1 = x1 + (ks2 + np.int32(1))
    x0, x1 = rounds(x0, x1, _ROTS[4:])
    x0 = x0 + ks2
    x1 = x1 + (ks0 + np.int32(2))
    x0, x1 = rounds(x0, x1, _ROTS[:4])
    x0 = x0 + ks0
    x1 = x1 + (ks1 + np.int32(3))
    x0, x1 = rounds(x0, x1, _ROTS[4:])
    x0 = x0 + ks1
    x1 = x1 + (ks2 + np.int32(4))
    x0, x1 = rounds(x0, x1, _ROTS[:4])
    x0 = x0 + ks2
    x1 = x1 + (ks0 + np.int32(5))
    return x0 ^ x1


def _weights_body(xt_ref, vt_ref, lw_ref, z_ref, anc_ref,
                  tab_ref, lsq_ref, xest_ref):
    xp = xt_ref[0] + DT * vt_ref[0]
    d2 = None
    for d in range(3):
        diff = xp[d:d + 1, :] - anc_ref[:, d:d + 1]
        sq = diff * diff
        d2 = sq if d2 is None else d2 + sq
    ranges = jnp.sqrt(d2)
    inn = z_ref[0] - ranges
    term = (inn * inn) / _VAR + _LOG_VAR + _LOG_2PI
    ll = np.float32(-0.5) * jnp.sum(term, axis=0, keepdims=True)

    lw = lw_ref[0] + ll
    m = jnp.max(lw, axis=1, keepdims=True)
    lse = jnp.log(jnp.sum(jnp.exp(lw - m), axis=1, keepdims=True)) + m
    lwn = lw - lse

    w_est = jnp.exp(lwn)
    x_est = jnp.sum(w_est * xp, axis=1, keepdims=True)

    m2 = jnp.max(lwn, axis=1, keepdims=True)
    e = jnp.exp(lwn - m2)
    w = e / jnp.sum(e, axis=1, keepdims=True)

    q = ALPHA * w + np.float32((1.0 - 0.5) * (1.0 / N))
    qs = jnp.sum(q, axis=1, keepdims=True)
    safe_q = jnp.where(qs > EPS, q / jnp.maximum(qs, EPS),
                       np.float32(1.0 / N))
    safe_q = jnp.maximum(safe_q, EPS)
    safe_q = safe_q / jnp.maximum(
        jnp.sum(safe_q, axis=1, keepdims=True), EPS)

    xest_ref[0] = x_est
    lsq_ref[0] = jnp.log(safe_q)
    tab_ref[0, 0:3, :] = xp
    tab_ref[0, 3:4, :] = w
    tab_ref[0, 4:5, :] = safe_q
    tab_ref[0, 5:8, :] = jnp.zeros((3, N), jnp.float32)


def _argmax_body(lsq_ref, out_ref):
    g = pl.program_id(0)
    lsq = lsq_ref[...]
    row = lax.broadcasted_iota(jnp.int32, (B, N), 0)
    col = lax.broadcasted_iota(jnp.int32, (B, N), 1)
    flat = row * np.int32(N) + col
    for d in range(D_CHUNK):
        base = (g * np.int32(D_CHUNK) + np.int32(d)) * np.int32(B * N)
        cnt = base + flat
        bits = _threefry_bits(cnt)
        fb = (lax.shift_right_logical(bits, np.int32(9))
              | np.int32(0x3F800000))
        f = lax.bitcast_convert_type(fb, jnp.float32) - np.float32(1.0)
        u = jnp.maximum(_TINY, f + _TINY)
        y = lsq - jnp.log(-jnp.log(u))
        m = jnp.max(y, axis=1, keepdims=True)
        cand = jnp.where(y == m, col, np.int32(N))
        out_ref[d * B:(d + 1) * B, :] = jnp.min(cand, axis=1, keepdims=True)


def _argmax_u_body(u_ref, lsq_ref, out_ref):
    lsq = lsq_ref[...]
    col = lax.broadcasted_iota(jnp.int32, (B, N), 1)
    for d in range(D_CHUNK):
        bits = u_ref[d * B:(d + 1) * B, :]
        fb = (lax.shift_right_logical(bits, np.int32(9))
              | np.int32(0x3F800000))
        f = lax.bitcast_convert_type(fb, jnp.float32) - np.float32(1.0)
        u = jnp.maximum(_TINY, f + _TINY)
        y = lsq - jnp.log(-jnp.log(u))
        m = jnp.max(y, axis=1, keepdims=True)
        cand = jnp.where(y == m, col, np.int32(N))
        out_ref[d * B:(d + 1) * B, :] = jnp.min(cand, axis=1, keepdims=True)


def _uniform_body(out_hbm, buf):
    c = lax.axis_index("c")
    s = lax.axis_index("s")
    wid = s * 2 + c
    row0 = wid * np.int32(ROWS_PW)
    base0 = np.int32((N - S_SC) * (B * N))
    iota16 = jnp.arange(16, dtype=jnp.int32)

    def tile_body(t, carry):
        rbase = row0 + t * np.int32(8)
        cbase = base0 + rbase * np.int32(N)
        for r in range(8):

            def vec_body(k, carry2):
                for j in range(4):
                    cnt = iota16 + (cbase + np.int32(r * N)
                                    + k * np.int32(64) + np.int32(j * 16))
                    buf[r, pl.ds(k * 64 + j * 16, 16)] = _threefry_bits(cnt)
                return carry2

            lax.fori_loop(0, N // 64, vec_body, 0)
        pltpu.sync_copy(buf, out_hbm.at[pl.ds(rbase, 8), :])
        return carry

    lax.fori_loop(0, TR_PW, tile_body, 0)


def _sc_uniform():
    mesh = plsc.VectorSubcoreMesh(core_axis_name="c", subcore_axis_name="s")
    kern = pl.kernel(
        _uniform_body,
        out_type=jax.ShapeDtypeStruct((SC_ROWS, N), jnp.int32),
        mesh=mesh,
        compiler_params=pltpu.CompilerParams(needs_layout_passes=False),
        scratch_types=[pltpu.VMEM((8, N), jnp.int32)],
    )
    return kern()


def _final_body(wsel_ref, qsel_ref, out_ref):
    wc = wsel_ref[...] / jnp.maximum(qsel_ref[...], EPS)
    lw = jnp.log(jnp.maximum(wc, EPS))
    m = jnp.max(lw, axis=1, keepdims=True)
    lse = jnp.log(jnp.sum(jnp.exp(lw - m), axis=1, keepdims=True)) + m
    out_ref[...] = lw - lse


def _gather_body(tab_hbm, idx_hbm, out_hbm, idx_v, tab_v,
                 o0, o1, o2, o3, o4):
    c = lax.axis_index("c")
    s = lax.axis_index("s")
    wid = s * 2 + c
    b = wid // 2
    h = wid % 2
    pltpu.sync_copy(idx_hbm.at[pl.ds(b * N + h * HALF, HALF)], idx_v)
    pltpu.sync_copy(tab_hbm.at[pl.ds(b * (8 * N), 8 * N)], tab_v)
    outs = (o0, o1, o2, o3, o4)

    def body(j, carry):
        iv = idx_v[pl.ds(j * 16, 16)]
        for r in range(5):
            vals = plsc.load_gather(tab_v, [iv + np.int32(r * N)])
            outs[r][pl.ds(j * 16, 16)] = vals
        return carry

    lax.fori_loop(0, HALF // 16, body, 0)
    for r in range(5):
        pltpu.sync_copy(
            outs[r],
            out_hbm.at[pl.ds((b * 5 + r) * N + h * HALF, HALF)])


def _sc_gather(tab, idx):
    mesh = plsc.VectorSubcoreMesh(core_axis_name="c", subcore_axis_name="s")
    kern = pl.kernel(
        _gather_body,
        out_type=jax.ShapeDtypeStruct((B * 5 * N,), jnp.float32),
        mesh=mesh,
        compiler_params=pltpu.CompilerParams(needs_layout_passes=False),
        scratch_types=[
            pltpu.VMEM((HALF,), jnp.int32),
            pltpu.VMEM((8 * N,), jnp.float32),
        ] + [pltpu.VMEM((HALF,), jnp.float32) for _ in range(5)],
    )
    return kern(tab.reshape(-1), idx.reshape(-1)).reshape(B, 5, N)


@jax.jit
def kernel(x_prev, v_prev, log_w_prev, z_t, anchors):
    xt = jnp.swapaxes(x_prev, 1, 2)
    vt = jnp.swapaxes(v_prev, 1, 2)
    lw3 = log_w_prev[:, None, :]
    z3 = z_t[:, :, None]

    tab, lsq3, xest3 = pl.pallas_call(
        _weights_body,
        grid=(B,),
        in_specs=[
            pl.BlockSpec((1, 3, N), lambda i: (i, 0, 0)),
            pl.BlockSpec((1, 3, N), lambda i: (i, 0, 0)),
            pl.BlockSpec((1, 1, N), lambda i: (i, 0, 0)),
            pl.BlockSpec((1, A, 1), lambda i: (i, 0, 0)),
            pl.BlockSpec((A, 3), lambda i: (0, 0)),
        ],
        out_specs=[
            pl.BlockSpec((1, 8, N), lambda i: (i, 0, 0)),
            pl.BlockSpec((1, 1, N), lambda i: (i, 0, 0)),
            pl.BlockSpec((1, 3, 1), lambda i: (i, 0, 0)),
        ],
        out_shape=[
            jax.ShapeDtypeStruct((B, 8, N), jnp.float32),
            jax.ShapeDtypeStruct((B, 1, N), jnp.float32),
            jax.ShapeDtypeStruct((B, 3, 1), jnp.float32),
        ],
    )(xt, vt, lw3, z3, anchors)

    lsq = lsq3[:, 0, :]

    u_sc = _sc_uniform()

    na_rows = (N - S_SC) * B
    idx_a = pl.pallas_call(
        _argmax_body,
        grid=(na_rows // ROWS,),
        in_specs=[pl.BlockSpec((B, N), lambda g: (0, 0))],
        out_specs=pl.BlockSpec((ROWS, 1), lambda g: (g, 0)),
        out_shape=jax.ShapeDtypeStruct((na_rows, 1), jnp.int32),
    )(lsq)

    idx_b = pl.pallas_call(
        _argmax_u_body,
        grid=(SC_ROWS // ROWS,),
        in_specs=[
            pl.BlockSpec((ROWS, N), lambda g: (g, 0)),
            pl.BlockSpec((B, N), lambda g: (0, 0)),
        ],
        out_specs=pl.BlockSpec((ROWS, 1), lambda g: (g, 0)),
        out_shape=jax.ShapeDtypeStruct((SC_ROWS, 1), jnp.int32),
    )(u_sc, lsq)

    idx_col = jnp.concatenate([idx_a, idx_b], axis=0)
    idx = idx_col.reshape(N, B).T

    gathered = _sc_gather(tab, idx)
    x_res = jnp.swapaxes(gathered[:, 0:3, :], 1, 2)
    w_sel = gathered[:, 3, :]
    q_sel = gathered[:, 4, :]

    log_w_res = pl.pallas_call(
        _final_body,
        in_specs=[
            pl.BlockSpec((B, N), lambda: (0, 0)),
            pl.BlockSpec((B, N), lambda: (0, 0)),
        ],
        out_specs=pl.BlockSpec((B, N), lambda: (0, 0)),
        out_shape=jax.ShapeDtypeStruct((B, N), jnp.float32),
    )(w_sel, q_sel)

    x_est = xest3.reshape(B, 3)
    return (x_res, log_w_res, x_est)


# speedup vs baseline: 1.0093x; 1.0093x over previous
import functools

import numpy as np
import jax
import jax.numpy as jnp
from jax import lax
from jax.experimental import pallas as pl
from jax.experimental.pallas import tpu as pltpu
from jax.experimental.pallas import tpu_sc as plsc

MIN_SCALE = np.float32(1e-06)
ALPHA = np.float32(0.5)
EPS = np.float32(1e-08)
OBS_SCALE = np.float32(0.5)
DT = np.float32(0.1)

B = 16
N = 4096
A = 64

_KEY = np.array([2247515013, 2545468385], dtype=np.uint32)
_KS = np.array(
    [_KEY[0], _KEY[1], _KEY[0] ^ _KEY[1] ^ np.uint32(0x1BD11BDA)],
    dtype=np.uint32,
).view(np.int32)
_ROTS = (13, 15, 26, 6, 17, 29, 16, 24)
_TINY = np.float32(np.finfo(np.float32).tiny)

_VAR = np.float32(max(OBS_SCALE * OBS_SCALE, MIN_SCALE))
_LOG_VAR = np.float32(np.log(_VAR))
_LOG_2PI = np.float32(np.log(np.float32(2.0 * np.pi)))

D_CHUNK = 16
ROWS = D_CHUNK * B
HALF = N // 2

S_SC = 1440
SC_ROWS = S_SC * B
ROWS_PW = SC_ROWS // 32
TR_PW = ROWS_PW // 8


def _rotl(x, r):
    return lax.shift_left(x, np.int32(r)) | lax.shift_right_logical(
        x, np.int32(32 - r))


def _threefry_bits(cnt):
    ks0, ks1, ks2 = (np.int32(_KS[0]), np.int32(_KS[1]), np.int32(_KS[2]))
    x0 = jnp.full(cnt.shape, ks0, jnp.int32)
    x1 = cnt + ks1

    def rounds(x0, x1, rots):
        for r in rots:
            x0 = x0 + x1
            x1 = _rotl(x1, r)
            x1 = x0 ^ x1
        return x0, x1

    x0, x1 = rounds(x0, x1, _ROTS[:4])
    x0 = x0 + ks1
    x1 = x1 + (ks2 + np.int32(1))
    x0, x1 = rounds(x0, x1, _ROTS[4:])
    x0 = x0 + ks2
    x1 = x1 + (ks0 + np.int32(2))
    x0, x1 = rounds(x0, x1, _ROTS[:4])
    x0 = x0 + ks0
    x1 = x1 + (ks1 + np.int32(3))
    x0, x1 = rounds(x0, x1, _ROTS[4:])
    x0 = x0 + ks1
    x1 = x1 + (ks2 + np.int32(4))
    x0, x1 = rounds(x0, x1, _ROTS[:4])
    x0 = x0 + ks2
    x1 = x1 + (ks0 + np.int32(5))
    return x0 ^ x1


def _weights_body(xt_ref, vt_ref, lw_ref, z_ref, anc_ref,
                  tab_ref, lsq_ref, xest_ref):
    xp = xt_ref[0] + DT * vt_ref[0]
    d2 = None
    for d in range(3):
        diff = xp[d:d + 1, :] - anc_ref[:, d:d + 1]
        sq = diff * diff
        d2 = sq if d2 is None else d2 + sq
    ranges = jnp.sqrt(d2)
    inn = z_ref[0] - ranges
    term = (inn * inn) / _VAR + _LOG_VAR + _LOG_2PI
    ll = np.float32(-0.5) * jnp.sum(term, axis=0, keepdims=True)

    lw = lw_ref[0] + ll
    m = jnp.max(lw, axis=1, keepdims=True)
    lse = jnp.log(jnp.sum(jnp.exp(lw - m), axis=1, keepdims=True)) + m
    lwn = lw - lse

    w_est = jnp.exp(lwn)
    x_est = jnp.sum(w_est * xp, axis=1, keepdims=True)

    m2 = jnp.max(lwn, axis=1, keepdims=True)
    e = jnp.exp(lwn - m2)
    w = e / jnp.sum(e, axis=1, keepdims=True)

    q = ALPHA * w + np.float32((1.0 - 0.5) * (1.0 / N))
    qs = jnp.sum(q, axis=1, keepdims=True)
    safe_q = jnp.where(qs > EPS, q / jnp.maximum(qs, EPS),
                       np.float32(1.0 / N))
    safe_q = jnp.maximum(safe_q, EPS)
    safe_q = safe_q / jnp.maximum(
        jnp.sum(safe_q, axis=1, keepdims=True), EPS)

    xest_ref[0] = x_est
    lsq_ref[0] = jnp.log(safe_q)
    tab_ref[0, 0:3, :] = xp
    tab_ref[0, 3:4, :] = w
    tab_ref[0, 4:5, :] = safe_q
    tab_ref[0, 5:8, :] = jnp.zeros((3, N), jnp.float32)


def _argmax_body(lsq_ref, out_ref):
    g = pl.program_id(0)
    lsq = lsq_ref[...]
    row = lax.broadcasted_iota(jnp.int32, (B, N), 0)
    col = lax.broadcasted_iota(jnp.int32, (B, N), 1)
    flat = row * np.int32(N) + col
    for d in range(D_CHUNK):
        base = (g * np.int32(D_CHUNK) + np.int32(d)) * np.int32(B * N)
        cnt = base + flat
        bits = _threefry_bits(cnt)
        fb = (lax.shift_right_logical(bits, np.int32(9))
              | np.int32(0x3F800000))
        f = lax.bitcast_convert_type(fb, jnp.float32) - np.float32(1.0)
        u = jnp.maximum(_TINY, f + _TINY)
        y = lsq - jnp.log(-jnp.log(u))
        m = jnp.max(y, axis=1, keepdims=True)
        cand = jnp.where(y == m, col, np.int32(N))
        out_ref[d * B:(d + 1) * B, :] = jnp.min(cand, axis=1, keepdims=True)


def _argmax_u_body(u_ref, lsq_ref, out_ref):
    lsq = lsq_ref[...]
    col = lax.broadcasted_iota(jnp.int32, (B, N), 1)
    for d in range(D_CHUNK):
        bits = u_ref[d * B:(d + 1) * B, :]
        fb = (lax.shift_right_logical(bits, np.int32(9))
              | np.int32(0x3F800000))
        f = lax.bitcast_convert_type(fb, jnp.float32) - np.float32(1.0)
        u = jnp.maximum(_TINY, f + _TINY)
        y = lsq - jnp.log(-jnp.log(u))
        m = jnp.max(y, axis=1, keepdims=True)
        cand = jnp.where(y == m, col, np.int32(N))
        out_ref[d * B:(d + 1) * B, :] = jnp.min(cand, axis=1, keepdims=True)


def _uniform_body(out_hbm, buf):
    c = lax.axis_index("c")
    s = lax.axis_index("s")
    wid = s * 2 + c
    row0 = wid * np.int32(ROWS_PW)
    base0 = np.int32((N - S_SC) * (B * N))
    iota16 = jnp.arange(16, dtype=jnp.int32)

    def tile_body(t, carry):
        rbase = row0 + t * np.int32(8)
        cbase = base0 + rbase * np.int32(N)
        for r in range(8):

            def vec_body(k, carry2):
                for j in range(4):
                    cnt = iota16 + (cbase + np.int32(r * N)
                                    + k * np.int32(64) + np.int32(j * 16))
                    buf[r, pl.ds(k * 64 + j * 16, 16)] = _threefry_bits(cnt)
                return carry2

            lax.fori_loop(0, N // 64, vec_body, 0)
        pltpu.sync_copy(buf, out_hbm.at[pl.ds(rbase, 8), :])
        return carry

    lax.fori_loop(0, TR_PW, tile_body, 0)


def _sc_uniform():
    mesh = plsc.VectorSubcoreMesh(core_axis_name="c", subcore_axis_name="s")
    kern = pl.kernel(
        _uniform_body,
        out_type=jax.ShapeDtypeStruct((SC_ROWS, N), jnp.int32),
        mesh=mesh,
        compiler_params=pltpu.CompilerParams(needs_layout_passes=False),
        scratch_types=[pltpu.VMEM((8, N), jnp.int32)],
    )
    return kern()


def _final_body(wsel_ref, qsel_ref, out_ref):
    wc = wsel_ref[...] / jnp.maximum(qsel_ref[...], EPS)
    lw = jnp.log(jnp.maximum(wc, EPS))
    m = jnp.max(lw, axis=1, keepdims=True)
    lse = jnp.log(jnp.sum(jnp.exp(lw - m), axis=1, keepdims=True)) + m
    out_ref[...] = lw - lse


def _gather_body(tab_hbm, idx_hbm, out_hbm, idx_v, tab_v,
                 o0, o1, o2, o3, o4):
    c = lax.axis_index("c")
    s = lax.axis_index("s")
    wid = s * 2 + c
    b = wid // 2
    h = wid % 2
    pltpu.sync_copy(idx_hbm.at[pl.ds(b * N + h * HALF, HALF)], idx_v)
    pltpu.sync_copy(tab_hbm.at[pl.ds(b * (8 * N), 8 * N)], tab_v)
    outs = (o0, o1, o2, o3, o4)

    def body(j, carry):
        iv = idx_v[pl.ds(j * 16, 16)]
        for r in range(5):
            vals = plsc.load_gather(tab_v, [iv + np.int32(r * N)])
            outs[r][pl.ds(j * 16, 16)] = vals
        return carry

    lax.fori_loop(0, HALF // 16, body, 0)
    for r in range(5):
        pltpu.sync_copy(
            outs[r],
            out_hbm.at[pl.ds((b * 5 + r) * N + h * HALF, HALF)])


def _sc_gather(tab, idx):
    mesh = plsc.VectorSubcoreMesh(core_axis_name="c", subcore_axis_name="s")
    kern = pl.kernel(
        _gather_body,
        out_type=jax.ShapeDtypeStruct((B * 5 * N,), jnp.float32),
        mesh=mesh,
        compiler_params=pltpu.CompilerParams(needs_layout_passes=False),
        scratch_types=[
            pltpu.VMEM((HALF,), jnp.int32),
            pltpu.VMEM((8 * N,), jnp.float32),
        ] + [pltpu.VMEM((HALF,), jnp.float32) for _ in range(5)],
    )
    return kern(tab.reshape(-1), idx.reshape(-1)).reshape(B, 5, N)


@jax.jit
def kernel(x_prev, v_prev, log_w_prev, z_t, anchors):
    xt = jnp.swapaxes(x_prev, 1, 2)
    vt = jnp.swapaxes(v_prev, 1, 2)
    lw3 = log_w_prev[:, None, :]
    z3 = z_t[:, :, None]

    tab, lsq3, xest3 = pl.pallas_call(
        _weights_body,
        grid=(B,),
        in_specs=[
            pl.BlockSpec((1, 3, N), lambda i: (i, 0, 0)),
            pl.BlockSpec((1, 3, N), lambda i: (i, 0, 0)),
            pl.BlockSpec((1, 1, N), lambda i: (i, 0, 0)),
            pl.BlockSpec((1, A, 1), lambda i: (i, 0, 0)),
            pl.BlockSpec((A, 3), lambda i: (0, 0)),
        ],
        out_specs=[
            pl.BlockSpec((1, 8, N), lambda i: (i, 0, 0)),
            pl.BlockSpec((1, 1, N), lambda i: (i, 0, 0)),
            pl.BlockSpec((1, 3, 1), lambda i: (i, 0, 0)),
        ],
        out_shape=[
            jax.ShapeDtypeStruct((B, 8, N), jnp.float32),
            jax.ShapeDtypeStruct((B, 1, N), jnp.float32),
            jax.ShapeDtypeStruct((B, 3, 1), jnp.float32),
        ],
    )(xt, vt, lw3, z3, anchors)

    lsq = lsq3[:, 0, :]

    u_sc = _sc_uniform()

    na_rows = (N - S_SC) * B
    idx_a = pl.pallas_call(
        _argmax_body,
        grid=(na_rows // ROWS,),
        in_specs=[pl.BlockSpec((B, N), lambda g: (0, 0))],
        out_specs=pl.BlockSpec((ROWS, 1), lambda g: (g, 0)),
        out_shape=jax.ShapeDtypeStruct((na_rows, 1), jnp.int32),
    )(lsq)

    idx_b = pl.pallas_call(
        _argmax_u_body,
        grid=(SC_ROWS // ROWS,),
        in_specs=[
            pl.BlockSpec((ROWS, N), lambda g: (g, 0)),
            pl.BlockSpec((B, N), lambda g: (0, 0)),
        ],
        out_specs=pl.BlockSpec((ROWS, 1), lambda g: (g, 0)),
        out_shape=jax.ShapeDtypeStruct((SC_ROWS, 1), jnp.int32),
    )(u_sc, lsq)

    idx_col = jnp.concatenate([idx_a, idx_b], axis=0)
    idx = idx_col.reshape(N, B).T

    gathered = _sc_gather(tab, idx)
    x_res = jnp.swapaxes(gathered[:, 0:3, :], 1, 2)
    w_sel = gathered[:, 3, :]
    q_sel = gathered[:, 4, :]

    log_w_res = pl.pallas_call(
        _final_body,
        in_specs=[
            pl.BlockSpec((B, N), lambda: (0, 0)),
            pl.BlockSpec((B, N), lambda: (0, 0)),
        ],
        out_specs=pl.BlockSpec((B, N), lambda: (0, 0)),
        out_shape=jax.ShapeDtypeStruct((B, N), jnp.float32),
    )(w_sel, q_sel)

    x_est = xest3.reshape(B, 3)
    return (x_res, log_w_res, x_est)


# speedup vs baseline: 1.0805x; 1.0705x over previous
import functools

import numpy as np
import jax
import jax.numpy as jnp
from jax import lax
from jax.experimental import pallas as pl
from jax.experimental.pallas import tpu as pltpu
from jax.experimental.pallas import tpu_sc as plsc

MIN_SCALE = np.float32(1e-06)
ALPHA = np.float32(0.5)
EPS = np.float32(1e-08)
OBS_SCALE = np.float32(0.5)
DT = np.float32(0.1)

B = 16
N = 4096
A = 64

_KEY = np.array([2247515013, 2545468385], dtype=np.uint32)
_KS = np.array(
    [_KEY[0], _KEY[1], _KEY[0] ^ _KEY[1] ^ np.uint32(0x1BD11BDA)],
    dtype=np.uint32,
).view(np.int32)
_ROTS = (13, 15, 26, 6, 17, 29, 16, 24)
_TINY = np.float32(np.finfo(np.float32).tiny)

_VAR = np.float32(max(OBS_SCALE * OBS_SCALE, MIN_SCALE))
_LOG_VAR = np.float32(np.log(_VAR))
_LOG_2PI = np.float32(np.log(np.float32(2.0 * np.pi)))

D_CHUNK = 16
ROWS = D_CHUNK * B
HALF = N // 2

S_SC = 1344
SC_ROWS = S_SC * B
ROWS_PW = SC_ROWS // 32
TR_PW = ROWS_PW // 8


def _rotl(x, r):
    return lax.shift_left(x, np.int32(r)) | lax.shift_right_logical(
        x, np.int32(32 - r))


def _threefry_bits(cnt):
    ks0, ks1, ks2 = (np.int32(_KS[0]), np.int32(_KS[1]), np.int32(_KS[2]))
    x0 = jnp.full(cnt.shape, ks0, jnp.int32)
    x1 = cnt + ks1

    def rounds(x0, x1, rots):
        for r in rots:
            x0 = x0 + x1
            x1 = _rotl(x1, r)
            x1 = x0 ^ x1
        return x0, x1

    x0, x1 = rounds(x0, x1, _ROTS[:4])
    x0 = x0 + ks1
    x1 = x1 + (ks2 + np.int32(1))
    x0, x1 = rounds(x0, x1, _ROTS[4:])
    x0 = x0 + ks2
    x1 = x1 + (ks0 + np.int32(2))
    x0, x1 = rounds(x0, x1, _ROTS[:4])
    x0 = x0 + ks0
    x1 = x1 + (ks1 + np.int32(3))
    x0, x1 = rounds(x0, x1, _ROTS[4:])
    x0 = x0 + ks1
    x1 = x1 + (ks2 + np.int32(4))
    x0, x1 = rounds(x0, x1, _ROTS[:4])
    x0 = x0 + ks2
    x1 = x1 + (ks0 + np.int32(5))
    return x0 ^ x1


def _weights_body(xt_ref, vt_ref, lw_ref, z_ref, anc_ref,
                  tab_ref, lsq_ref, xest_ref):
    xp = xt_ref[0] + DT * vt_ref[0]
    d2 = None
    for d in range(3):
        diff = xp[d:d + 1, :] - anc_ref[:, d:d + 1]
        sq = diff * diff
        d2 = sq if d2 is None else d2 + sq
    ranges = jnp.sqrt(d2)
    inn = z_ref[0] - ranges
    term = (inn * inn) / _VAR + _LOG_VAR + _LOG_2PI
    ll = np.float32(-0.5) * jnp.sum(term, axis=0, keepdims=True)

    lw = lw_ref[0] + ll
    m = jnp.max(lw, axis=1, keepdims=True)
    lse = jnp.log(jnp.sum(jnp.exp(lw - m), axis=1, keepdims=True)) + m
    lwn = lw - lse

    w_est = jnp.exp(lwn)
    x_est = jnp.sum(w_est * xp, axis=1, keepdims=True)

    m2 = jnp.max(lwn, axis=1, keepdims=True)
    e = jnp.exp(lwn - m2)
    w = e / jnp.sum(e, axis=1, keepdims=True)

    q = ALPHA * w + np.float32((1.0 - 0.5) * (1.0 / N))
    qs = jnp.sum(q, axis=1, keepdims=True)
    safe_q = jnp.where(qs > EPS, q / jnp.maximum(qs, EPS),
                       np.float32(1.0 / N))
    safe_q = jnp.maximum(safe_q, EPS)
    safe_q = safe_q / jnp.maximum(
        jnp.sum(safe_q, axis=1, keepdims=True), EPS)

    xest_ref[0] = x_est
    lsq_ref[0] = jnp.log(safe_q)
    tab_ref[0, 0:3, :] = xp
    tab_ref[0, 3:4, :] = w
    tab_ref[0, 4:5, :] = safe_q
    tab_ref[0, 5:8, :] = jnp.zeros((3, N), jnp.float32)


def _argmax_body(lsq_ref, out_ref):
    g = pl.program_id(0)
    lsq = lsq_ref[...]
    row = lax.broadcasted_iota(jnp.int32, (B, N), 0)
    col = lax.broadcasted_iota(jnp.int32, (B, N), 1)
    flat = row * np.int32(N) + col
    for d in range(D_CHUNK):
        base = (g * np.int32(D_CHUNK) + np.int32(d)) * np.int32(B * N)
        cnt = base + flat
        bits = _threefry_bits(cnt)
        fb = (lax.shift_right_logical(bits, np.int32(9))
              | np.int32(0x3F800000))
        f = lax.bitcast_convert_type(fb, jnp.float32) - np.float32(1.0)
        u = jnp.maximum(_TINY, f + _TINY)
        y = lsq - jnp.log(-jnp.log(u))
        m = jnp.max(y, axis=1, keepdims=True)
        cand = jnp.where(y == m, col, np.int32(N))
        out_ref[d * B:(d + 1) * B, :] = jnp.min(cand, axis=1, keepdims=True)


def _argmax_u_body(u_ref, lsq_ref, out_ref):
    lsq = lsq_ref[...]
    col = lax.broadcasted_iota(jnp.int32, (B, N), 1)
    for d in range(D_CHUNK):
        bits = u_ref[d * B:(d + 1) * B, :]
        fb = (lax.shift_right_logical(bits, np.int32(9))
              | np.int32(0x3F800000))
        f = lax.bitcast_convert_type(fb, jnp.float32) - np.float32(1.0)
        u = jnp.maximum(_TINY, f + _TINY)
        y = lsq - jnp.log(-jnp.log(u))
        m = jnp.max(y, axis=1, keepdims=True)
        cand = jnp.where(y == m, col, np.int32(N))
        out_ref[d * B:(d + 1) * B, :] = jnp.min(cand, axis=1, keepdims=True)


def _uniform_body(out_hbm, buf0):
    c = lax.axis_index("c")
    s = lax.axis_index("s")
    wid = s * 2 + c
    row0 = wid * np.int32(ROWS_PW)
    base0 = np.int32((N - S_SC) * (B * N))
    iota16 = jnp.arange(16, dtype=jnp.int32)

    def fill(buf, t):
        cbase = base0 + (row0 + t * np.int32(8)) * np.int32(N)
        for r in range(8):

            def vec_body(k, carry2):
                for j in range(4):
                    cnt = iota16 + (cbase + np.int32(r * N)
                                    + k * np.int32(64) + np.int32(j * 16))
                    buf[r, pl.ds(k * 64 + j * 16, 16)] = _threefry_bits(cnt)
                return carry2

            lax.fori_loop(0, N // 64, vec_body, 0)

    def tile_body(t, carry):
        rbase = row0 + t * np.int32(8)
        fill(buf0, t)
        pltpu.sync_copy(buf0, out_hbm.at[pl.ds(rbase, 8), :])
        return carry

    lax.fori_loop(0, TR_PW, tile_body, 0)


def _sc_uniform():
    mesh = plsc.VectorSubcoreMesh(core_axis_name="c", subcore_axis_name="s")
    kern = pl.kernel(
        _uniform_body,
        out_type=jax.ShapeDtypeStruct((SC_ROWS, N), jnp.int32),
        mesh=mesh,
        compiler_params=pltpu.CompilerParams(needs_layout_passes=False),
        scratch_types=[pltpu.VMEM((8, N), jnp.int32)],
    )
    return kern()


def _final_body(wsel_ref, qsel_ref, out_ref):
    wc = wsel_ref[...] / jnp.maximum(qsel_ref[...], EPS)
    lw = jnp.log(jnp.maximum(wc, EPS))
    m = jnp.max(lw, axis=1, keepdims=True)
    lse = jnp.log(jnp.sum(jnp.exp(lw - m), axis=1, keepdims=True)) + m
    out_ref[...] = lw - lse


def _gather_body(tab_hbm, idx_hbm, out_hbm, idx_v, tab_v,
                 o0, o1, o2, o3, o4):
    c = lax.axis_index("c")
    s = lax.axis_index("s")
    wid = s * 2 + c
    b = wid // 2
    h = wid % 2
    pltpu.sync_copy(idx_hbm.at[pl.ds(b * N + h * HALF, HALF)], idx_v)
    pltpu.sync_copy(tab_hbm.at[pl.ds(b * (8 * N), 8 * N)], tab_v)
    outs = (o0, o1, o2, o3, o4)

    def body(j, carry):
        iv = idx_v[pl.ds(j * 16, 16)]
        for r in range(5):
            vals = plsc.load_gather(tab_v, [iv + np.int32(r * N)])
            outs[r][pl.ds(j * 16, 16)] = vals
        return carry

    lax.fori_loop(0, HALF // 16, body, 0)
    for r in range(5):
        pltpu.sync_copy(
            outs[r],
            out_hbm.at[pl.ds((b * 5 + r) * N + h * HALF, HALF)])


def _sc_gather(tab, idx):
    mesh = plsc.VectorSubcoreMesh(core_axis_name="c", subcore_axis_name="s")
    kern = pl.kernel(
        _gather_body,
        out_type=jax.ShapeDtypeStruct((B * 5 * N,), jnp.float32),
        mesh=mesh,
        compiler_params=pltpu.CompilerParams(needs_layout_passes=False),
        scratch_types=[
            pltpu.VMEM((HALF,), jnp.int32),
            pltpu.VMEM((8 * N,), jnp.float32),
        ] + [pltpu.VMEM((HALF,), jnp.float32) for _ in range(5)],
    )
    return kern(tab.reshape(-1), idx.reshape(-1)).reshape(B, 5, N)


@jax.jit
def kernel(x_prev, v_prev, log_w_prev, z_t, anchors):
    xt = jnp.swapaxes(x_prev, 1, 2)
    vt = jnp.swapaxes(v_prev, 1, 2)
    lw3 = log_w_prev[:, None, :]
    z3 = z_t[:, :, None]

    tab, lsq3, xest3 = pl.pallas_call(
        _weights_body,
        grid=(B,),
        in_specs=[
            pl.BlockSpec((1, 3, N), lambda i: (i, 0, 0)),
            pl.BlockSpec((1, 3, N), lambda i: (i, 0, 0)),
            pl.BlockSpec((1, 1, N), lambda i: (i, 0, 0)),
            pl.BlockSpec((1, A, 1), lambda i: (i, 0, 0)),
            pl.BlockSpec((A, 3), lambda i: (0, 0)),
        ],
        out_specs=[
            pl.BlockSpec((1, 8, N), lambda i: (i, 0, 0)),
            pl.BlockSpec((1, 1, N), lambda i: (i, 0, 0)),
            pl.BlockSpec((1, 3, 1), lambda i: (i, 0, 0)),
        ],
        out_shape=[
            jax.ShapeDtypeStruct((B, 8, N), jnp.float32),
            jax.ShapeDtypeStruct((B, 1, N), jnp.float32),
            jax.ShapeDtypeStruct((B, 3, 1), jnp.float32),
        ],
    )(xt, vt, lw3, z3, anchors)

    lsq = lsq3[:, 0, :]

    u_sc = _sc_uniform()

    na_rows = (N - S_SC) * B
    idx_a = pl.pallas_call(
        _argmax_body,
        grid=(na_rows // ROWS,),
        in_specs=[pl.BlockSpec((B, N), lambda g: (0, 0))],
        out_specs=pl.BlockSpec((ROWS, 1), lambda g: (g, 0)),
        out_shape=jax.ShapeDtypeStruct((na_rows, 1), jnp.int32),
    )(lsq)

    idx_b = pl.pallas_call(
        _argmax_u_body,
        grid=(SC_ROWS // ROWS,),
        in_specs=[
            pl.BlockSpec((ROWS, N), lambda g: (g, 0)),
            pl.BlockSpec((B, N), lambda g: (0, 0)),
        ],
        out_specs=pl.BlockSpec((ROWS, 1), lambda g: (g, 0)),
        out_shape=jax.ShapeDtypeStruct((SC_ROWS, 1), jnp.int32),
    )(u_sc, lsq)

    idx_col = jnp.concatenate([idx_a, idx_b], axis=0)
    idx = idx_col.reshape(N, B).T

    gathered = _sc_gather(tab, idx)
    x_res = jnp.swapaxes(gathered[:, 0:3, :], 1, 2)
    w_sel = gathered[:, 3, :]
    q_sel = gathered[:, 4, :]

    log_w_res = pl.pallas_call(
        _final_body,
        in_specs=[
            pl.BlockSpec((B, N), lambda: (0, 0)),
            pl.BlockSpec((B, N), lambda: (0, 0)),
        ],
        out_specs=pl.BlockSpec((B, N), lambda: (0, 0)),
        out_shape=jax.ShapeDtypeStruct((B, N), jnp.float32),
    )(w_sel, q_sel)

    x_est = xest3.reshape(B, 3)
    return (x_res, log_w_res, x_est)


# speedup vs baseline: 1.1063x; 1.0240x over previous
import functools

import numpy as np
import jax
import jax.numpy as jnp
from jax import lax
from jax.experimental import pallas as pl
from jax.experimental.pallas import tpu as pltpu
from jax.experimental.pallas import tpu_sc as plsc

MIN_SCALE = np.float32(1e-06)
ALPHA = np.float32(0.5)
EPS = np.float32(1e-08)
OBS_SCALE = np.float32(0.5)
DT = np.float32(0.1)

B = 16
N = 4096
A = 64

_KEY = np.array([2247515013, 2545468385], dtype=np.uint32)
_KS = np.array(
    [_KEY[0], _KEY[1], _KEY[0] ^ _KEY[1] ^ np.uint32(0x1BD11BDA)],
    dtype=np.uint32,
).view(np.int32)
_ROTS = (13, 15, 26, 6, 17, 29, 16, 24)
_TINY = np.float32(np.finfo(np.float32).tiny)

_VAR = np.float32(max(OBS_SCALE * OBS_SCALE, MIN_SCALE))
_LOG_VAR = np.float32(np.log(_VAR))
_LOG_2PI = np.float32(np.log(np.float32(2.0 * np.pi)))

D_CHUNK = 16
ROWS = D_CHUNK * B
HALF = N // 2

S_SC = 1216
SC_ROWS = S_SC * B
ROWS_PW = SC_ROWS // 32
TR_PW = ROWS_PW // 8


def _rotl(x, r):
    return lax.shift_left(x, np.int32(r)) | lax.shift_right_logical(
        x, np.int32(32 - r))


def _threefry_bits(cnt):
    ks0, ks1, ks2 = (np.int32(_KS[0]), np.int32(_KS[1]), np.int32(_KS[2]))
    x0 = jnp.full(cnt.shape, ks0, jnp.int32)
    x1 = cnt + ks1

    def rounds(x0, x1, rots):
        for r in rots:
            x0 = x0 + x1
            x1 = _rotl(x1, r)
            x1 = x0 ^ x1
        return x0, x1

    x0, x1 = rounds(x0, x1, _ROTS[:4])
    x0 = x0 + ks1
    x1 = x1 + (ks2 + np.int32(1))
    x0, x1 = rounds(x0, x1, _ROTS[4:])
    x0 = x0 + ks2
    x1 = x1 + (ks0 + np.int32(2))
    x0, x1 = rounds(x0, x1, _ROTS[:4])
    x0 = x0 + ks0
    x1 = x1 + (ks1 + np.int32(3))
    x0, x1 = rounds(x0, x1, _ROTS[4:])
    x0 = x0 + ks1
    x1 = x1 + (ks2 + np.int32(4))
    x0, x1 = rounds(x0, x1, _ROTS[:4])
    x0 = x0 + ks2
    x1 = x1 + (ks0 + np.int32(5))
    return x0 ^ x1


def _weights_body(xt_ref, vt_ref, lw_ref, z_ref, anc_ref,
                  tab_ref, lsq_ref, xest_ref):
    xp = xt_ref[0] + DT * vt_ref[0]
    d2 = None
    for d in range(3):
        diff = xp[d:d + 1, :] - anc_ref[:, d:d + 1]
        sq = diff * diff
        d2 = sq if d2 is None else d2 + sq
    ranges = jnp.sqrt(d2)
    inn = z_ref[0] - ranges
    term = (inn * inn) / _VAR + _LOG_VAR + _LOG_2PI
    ll = np.float32(-0.5) * jnp.sum(term, axis=0, keepdims=True)

    lw = lw_ref[0] + ll
    m = jnp.max(lw, axis=1, keepdims=True)
    lse = jnp.log(jnp.sum(jnp.exp(lw - m), axis=1, keepdims=True)) + m
    lwn = lw - lse

    w_est = jnp.exp(lwn)
    x_est = jnp.sum(w_est * xp, axis=1, keepdims=True)

    m2 = jnp.max(lwn, axis=1, keepdims=True)
    e = jnp.exp(lwn - m2)
    w = e / jnp.sum(e, axis=1, keepdims=True)

    q = ALPHA * w + np.float32((1.0 - 0.5) * (1.0 / N))
    qs = jnp.sum(q, axis=1, keepdims=True)
    safe_q = jnp.where(qs > EPS, q / jnp.maximum(qs, EPS),
                       np.float32(1.0 / N))
    safe_q = jnp.maximum(safe_q, EPS)
    safe_q = safe_q / jnp.maximum(
        jnp.sum(safe_q, axis=1, keepdims=True), EPS)

    xest_ref[0] = x_est
    lsq_ref[0] = jnp.log(safe_q)
    tab_ref[0, 0:3, :] = xp
    tab_ref[0, 3:4, :] = w
    tab_ref[0, 4:5, :] = safe_q
    tab_ref[0, 5:8, :] = jnp.zeros((3, N), jnp.float32)


def _argmax_body(lsq_ref, out_ref):
    g = pl.program_id(0)
    lsq = lsq_ref[...]
    row = lax.broadcasted_iota(jnp.int32, (B, N), 0)
    col = lax.broadcasted_iota(jnp.int32, (B, N), 1)
    flat = row * np.int32(N) + col
    for d in range(D_CHUNK):
        base = (g * np.int32(D_CHUNK) + np.int32(d)) * np.int32(B * N)
        cnt = base + flat
        bits = _threefry_bits(cnt)
        fb = (lax.shift_right_logical(bits, np.int32(9))
              | np.int32(0x3F800000))
        f = lax.bitcast_convert_type(fb, jnp.float32) - np.float32(1.0)
        u = jnp.maximum(_TINY, f + _TINY)
        y = lsq - jnp.log(-jnp.log(u))
        m = jnp.max(y, axis=1, keepdims=True)
        cand = jnp.where(y == m, col, np.int32(N))
        out_ref[d * B:(d + 1) * B, :] = jnp.min(cand, axis=1, keepdims=True)


def _argmax_u_body(u_ref, lsq_ref, out_ref):
    lsq = lsq_ref[...]
    col = lax.broadcasted_iota(jnp.int32, (B, N), 1)
    for d in range(D_CHUNK):
        bits = u_ref[d * B:(d + 1) * B, :]
        fb = (lax.shift_right_logical(bits, np.int32(9))
              | np.int32(0x3F800000))
        f = lax.bitcast_convert_type(fb, jnp.float32) - np.float32(1.0)
        u = jnp.maximum(_TINY, f + _TINY)
        y = lsq - jnp.log(-jnp.log(u))
        m = jnp.max(y, axis=1, keepdims=True)
        cand = jnp.where(y == m, col, np.int32(N))
        out_ref[d * B:(d + 1) * B, :] = jnp.min(cand, axis=1, keepdims=True)


def _uniform_body(out_hbm, buf0):
    c = lax.axis_index("c")
    s = lax.axis_index("s")
    wid = s * 2 + c
    row0 = wid * np.int32(ROWS_PW)
    base0 = np.int32((N - S_SC) * (B * N))
    iota16 = jnp.arange(16, dtype=jnp.int32)

    def fill(buf, t):
        cbase = base0 + (row0 + t * np.int32(8)) * np.int32(N)
        for r in range(8):

            def vec_body(k, carry2):
                for j in range(4):
                    cnt = iota16 + (cbase + np.int32(r * N)
                                    + k * np.int32(64) + np.int32(j * 16))
                    buf[r, pl.ds(k * 64 + j * 16, 16)] = _threefry_bits(cnt)
                return carry2

            lax.fori_loop(0, N // 64, vec_body, 0)

    def tile_body(t, carry):
        rbase = row0 + t * np.int32(8)
        fill(buf0, t)
        pltpu.sync_copy(buf0, out_hbm.at[pl.ds(rbase, 8), :])
        return carry

    lax.fori_loop(0, TR_PW, tile_body, 0)


def _sc_uniform():
    mesh = plsc.VectorSubcoreMesh(core_axis_name="c", subcore_axis_name="s")
    kern = pl.kernel(
        _uniform_body,
        out_type=jax.ShapeDtypeStruct((SC_ROWS, N), jnp.int32),
        mesh=mesh,
        compiler_params=pltpu.CompilerParams(needs_layout_passes=False),
        scratch_types=[pltpu.VMEM((8, N), jnp.int32)],
    )
    return kern()


def _final_body(wsel_ref, qsel_ref, out_ref):
    wc = wsel_ref[...] / jnp.maximum(qsel_ref[...], EPS)
    lw = jnp.log(jnp.maximum(wc, EPS))
    m = jnp.max(lw, axis=1, keepdims=True)
    lse = jnp.log(jnp.sum(jnp.exp(lw - m), axis=1, keepdims=True)) + m
    out_ref[...] = lw - lse


def _gather_body(tab_hbm, idx_hbm, out_hbm, idx_v, tab_v,
                 o0, o1, o2, o3, o4):
    c = lax.axis_index("c")
    s = lax.axis_index("s")
    wid = s * 2 + c
    b = wid // 2
    h = wid % 2
    pltpu.sync_copy(idx_hbm.at[pl.ds(b * N + h * HALF, HALF)], idx_v)
    pltpu.sync_copy(tab_hbm.at[pl.ds(b * (8 * N), 8 * N)], tab_v)
    outs = (o0, o1, o2, o3, o4)

    def body(j, carry):
        iv = idx_v[pl.ds(j * 16, 16)]
        for r in range(5):
            vals = plsc.load_gather(tab_v, [iv + np.int32(r * N)])
            outs[r][pl.ds(j * 16, 16)] = vals
        return carry

    lax.fori_loop(0, HALF // 16, body, 0)
    for r in range(5):
        pltpu.sync_copy(
            outs[r],
            out_hbm.at[pl.ds((b * 5 + r) * N + h * HALF, HALF)])


def _sc_gather(tab, idx):
    mesh = plsc.VectorSubcoreMesh(core_axis_name="c", subcore_axis_name="s")
    kern = pl.kernel(
        _gather_body,
        out_type=jax.ShapeDtypeStruct((B * 5 * N,), jnp.float32),
        mesh=mesh,
        compiler_params=pltpu.CompilerParams(needs_layout_passes=False),
        scratch_types=[
            pltpu.VMEM((HALF,), jnp.int32),
            pltpu.VMEM((8 * N,), jnp.float32),
        ] + [pltpu.VMEM((HALF,), jnp.float32) for _ in range(5)],
    )
    return kern(tab.reshape(-1), idx.reshape(-1)).reshape(B, 5, N)


@jax.jit
def kernel(x_prev, v_prev, log_w_prev, z_t, anchors):
    xt = jnp.swapaxes(x_prev, 1, 2)
    vt = jnp.swapaxes(v_prev, 1, 2)
    lw3 = log_w_prev[:, None, :]
    z3 = z_t[:, :, None]

    tab, lsq3, xest3 = pl.pallas_call(
        _weights_body,
        grid=(B,),
        in_specs=[
            pl.BlockSpec((1, 3, N), lambda i: (i, 0, 0)),
            pl.BlockSpec((1, 3, N), lambda i: (i, 0, 0)),
            pl.BlockSpec((1, 1, N), lambda i: (i, 0, 0)),
            pl.BlockSpec((1, A, 1), lambda i: (i, 0, 0)),
            pl.BlockSpec((A, 3), lambda i: (0, 0)),
        ],
        out_specs=[
            pl.BlockSpec((1, 8, N), lambda i: (i, 0, 0)),
            pl.BlockSpec((1, 1, N), lambda i: (i, 0, 0)),
            pl.BlockSpec((1, 3, 1), lambda i: (i, 0, 0)),
        ],
        out_shape=[
            jax.ShapeDtypeStruct((B, 8, N), jnp.float32),
            jax.ShapeDtypeStruct((B, 1, N), jnp.float32),
            jax.ShapeDtypeStruct((B, 3, 1), jnp.float32),
        ],
    )(xt, vt, lw3, z3, anchors)

    lsq = lsq3[:, 0, :]

    u_sc = _sc_uniform()

    na_rows = (N - S_SC) * B
    idx_a = pl.pallas_call(
        _argmax_body,
        grid=(na_rows // ROWS,),
        in_specs=[pl.BlockSpec((B, N), lambda g: (0, 0))],
        out_specs=pl.BlockSpec((ROWS, 1), lambda g: (g, 0)),
        out_shape=jax.ShapeDtypeStruct((na_rows, 1), jnp.int32),
    )(lsq)

    idx_b = pl.pallas_call(
        _argmax_u_body,
        grid=(SC_ROWS // ROWS,),
        in_specs=[
            pl.BlockSpec((ROWS, N), lambda g: (g, 0)),
            pl.BlockSpec((B, N), lambda g: (0, 0)),
        ],
        out_specs=pl.BlockSpec((ROWS, 1), lambda g: (g, 0)),
        out_shape=jax.ShapeDtypeStruct((SC_ROWS, 1), jnp.int32),
    )(u_sc, lsq)

    idx_col = jnp.concatenate([idx_a, idx_b], axis=0)
    idx = idx_col.reshape(N, B).T

    gathered = _sc_gather(tab, idx)
    x_res = jnp.swapaxes(gathered[:, 0:3, :], 1, 2)
    w_sel = gathered[:, 3, :]
    q_sel = gathered[:, 4, :]

    log_w_res = pl.pallas_call(
        _final_body,
        in_specs=[
            pl.BlockSpec((B, N), lambda: (0, 0)),
            pl.BlockSpec((B, N), lambda: (0, 0)),
        ],
        out_specs=pl.BlockSpec((B, N), lambda: (0, 0)),
        out_shape=jax.ShapeDtypeStruct((B, N), jnp.float32),
    )(w_sel, q_sel)

    x_est = xest3.reshape(B, 3)
    return (x_res, log_w_res, x_est)


# speedup vs baseline: 1.1268x; 1.0185x over previous
import functools

import numpy as np
import jax
import jax.numpy as jnp
from jax import lax
from jax.experimental import pallas as pl
from jax.experimental.pallas import tpu as pltpu
from jax.experimental.pallas import tpu_sc as plsc

MIN_SCALE = np.float32(1e-06)
ALPHA = np.float32(0.5)
EPS = np.float32(1e-08)
OBS_SCALE = np.float32(0.5)
DT = np.float32(0.1)

B = 16
N = 4096
A = 64

_KEY = np.array([2247515013, 2545468385], dtype=np.uint32)
_KS = np.array(
    [_KEY[0], _KEY[1], _KEY[0] ^ _KEY[1] ^ np.uint32(0x1BD11BDA)],
    dtype=np.uint32,
).view(np.int32)
_ROTS = (13, 15, 26, 6, 17, 29, 16, 24)
_TINY = np.float32(np.finfo(np.float32).tiny)

_VAR = np.float32(max(OBS_SCALE * OBS_SCALE, MIN_SCALE))
_LOG_VAR = np.float32(np.log(_VAR))
_LOG_2PI = np.float32(np.log(np.float32(2.0 * np.pi)))

D_CHUNK = 16
ROWS = D_CHUNK * B
HALF = N // 2

S_SC = 1280
SC_ROWS = S_SC * B
ROWS_PW = SC_ROWS // 32
TR_PW = ROWS_PW // 8


def _rotl(x, r):
    return lax.shift_left(x, np.int32(r)) | lax.shift_right_logical(
        x, np.int32(32 - r))


def _threefry_bits(cnt):
    ks0, ks1, ks2 = (np.int32(_KS[0]), np.int32(_KS[1]), np.int32(_KS[2]))
    x0 = jnp.full(cnt.shape, ks0, jnp.int32)
    x1 = cnt + ks1

    def rounds(x0, x1, rots):
        for r in rots:
            x0 = x0 + x1
            x1 = _rotl(x1, r)
            x1 = x0 ^ x1
        return x0, x1

    x0, x1 = rounds(x0, x1, _ROTS[:4])
    x0 = x0 + ks1
    x1 = x1 + (ks2 + np.int32(1))
    x0, x1 = rounds(x0, x1, _ROTS[4:])
    x0 = x0 + ks2
    x1 = x1 + (ks0 + np.int32(2))
    x0, x1 = rounds(x0, x1, _ROTS[:4])
    x0 = x0 + ks0
    x1 = x1 + (ks1 + np.int32(3))
    x0, x1 = rounds(x0, x1, _ROTS[4:])
    x0 = x0 + ks1
    x1 = x1 + (ks2 + np.int32(4))
    x0, x1 = rounds(x0, x1, _ROTS[:4])
    x0 = x0 + ks2
    x1 = x1 + (ks0 + np.int32(5))
    return x0 ^ x1


def _weights_body(xt_ref, vt_ref, lw_ref, z_ref, anc_ref,
                  tab_ref, lsq_ref, xest_ref):
    xp = xt_ref[0] + DT * vt_ref[0]
    d2 = None
    for d in range(3):
        diff = xp[d:d + 1, :] - anc_ref[:, d:d + 1]
        sq = diff * diff
        d2 = sq if d2 is None else d2 + sq
    ranges = jnp.sqrt(d2)
    inn = z_ref[0] - ranges
    term = (inn * inn) / _VAR + _LOG_VAR + _LOG_2PI
    ll = np.float32(-0.5) * jnp.sum(term, axis=0, keepdims=True)

    lw = lw_ref[0] + ll
    m = jnp.max(lw, axis=1, keepdims=True)
    lse = jnp.log(jnp.sum(jnp.exp(lw - m), axis=1, keepdims=True)) + m
    lwn = lw - lse

    w_est = jnp.exp(lwn)
    x_est = jnp.sum(w_est * xp, axis=1, keepdims=True)

    m2 = jnp.max(lwn, axis=1, keepdims=True)
    e = jnp.exp(lwn - m2)
    w = e / jnp.sum(e, axis=1, keepdims=True)

    q = ALPHA * w + np.float32((1.0 - 0.5) * (1.0 / N))
    qs = jnp.sum(q, axis=1, keepdims=True)
    safe_q = jnp.where(qs > EPS, q / jnp.maximum(qs, EPS),
                       np.float32(1.0 / N))
    safe_q = jnp.maximum(safe_q, EPS)
    safe_q = safe_q / jnp.maximum(
        jnp.sum(safe_q, axis=1, keepdims=True), EPS)

    xest_ref[0] = x_est
    lsq_ref[0] = jnp.log(safe_q)
    tab_ref[0, 0:3, :] = xp
    tab_ref[0, 3:4, :] = w
    tab_ref[0, 4:5, :] = safe_q
    tab_ref[0, 5:8, :] = jnp.zeros((3, N), jnp.float32)


def _argmax_body(lsq_ref, out_ref):
    g = pl.program_id(0)
    lsq = lsq_ref[...]
    row = lax.broadcasted_iota(jnp.int32, (B, N), 0)
    col = lax.broadcasted_iota(jnp.int32, (B, N), 1)
    flat = row * np.int32(N) + col
    for d in range(D_CHUNK):
        base = (g * np.int32(D_CHUNK) + np.int32(d)) * np.int32(B * N)
        cnt = base + flat
        bits = _threefry_bits(cnt)
        fb = (lax.shift_right_logical(bits, np.int32(9))
              | np.int32(0x3F800000))
        f = lax.bitcast_convert_type(fb, jnp.float32) - np.float32(1.0)
        u = jnp.maximum(_TINY, f + _TINY)
        y = lsq - jnp.log(-jnp.log(u))
        m = jnp.max(y, axis=1, keepdims=True)
        cand = jnp.where(y == m, col, np.int32(N))
        out_ref[d * B:(d + 1) * B, :] = jnp.min(cand, axis=1, keepdims=True)


def _argmax_u_body(u_ref, lsq_ref, out_ref):
    lsq = lsq_ref[...]
    col = lax.broadcasted_iota(jnp.int32, (B, N), 1)
    for d in range(D_CHUNK):
        bits = u_ref[d * B:(d + 1) * B, :]
        fb = (lax.shift_right_logical(bits, np.int32(9))
              | np.int32(0x3F800000))
        f = lax.bitcast_convert_type(fb, jnp.float32) - np.float32(1.0)
        u = jnp.maximum(_TINY, f + _TINY)
        y = lsq - jnp.log(-jnp.log(u))
        m = jnp.max(y, axis=1, keepdims=True)
        cand = jnp.where(y == m, col, np.int32(N))
        out_ref[d * B:(d + 1) * B, :] = jnp.min(cand, axis=1, keepdims=True)


def _uniform_body(out_hbm, buf0):
    c = lax.axis_index("c")
    s = lax.axis_index("s")
    wid = s * 2 + c
    row0 = wid * np.int32(ROWS_PW)
    base0 = np.int32((N - S_SC) * (B * N))
    iota16 = jnp.arange(16, dtype=jnp.int32)

    def fill(buf, t):
        cbase = base0 + (row0 + t * np.int32(8)) * np.int32(N)
        for r in range(8):

            def vec_body(k, carry2):
                for j in range(4):
                    cnt = iota16 + (cbase + np.int32(r * N)
                                    + k * np.int32(64) + np.int32(j * 16))
                    buf[r, pl.ds(k * 64 + j * 16, 16)] = _threefry_bits(cnt)
                return carry2

            lax.fori_loop(0, N // 64, vec_body, 0)

    def tile_body(t, carry):
        rbase = row0 + t * np.int32(8)
        fill(buf0, t)
        pltpu.sync_copy(buf0, out_hbm.at[pl.ds(rbase, 8), :])
        return carry

    lax.fori_loop(0, TR_PW, tile_body, 0)


def _sc_uniform():
    mesh = plsc.VectorSubcoreMesh(core_axis_name="c", subcore_axis_name="s")
    kern = pl.kernel(
        _uniform_body,
        out_type=jax.ShapeDtypeStruct((SC_ROWS, N), jnp.int32),
        mesh=mesh,
        compiler_params=pltpu.CompilerParams(needs_layout_passes=False),
        scratch_types=[pltpu.VMEM((8, N), jnp.int32)],
    )
    return kern()


def _final_body(wsel_ref, qsel_ref, out_ref):
    wc = wsel_ref[...] / jnp.maximum(qsel_ref[...], EPS)
    lw = jnp.log(jnp.maximum(wc, EPS))
    m = jnp.max(lw, axis=1, keepdims=True)
    lse = jnp.log(jnp.sum(jnp.exp(lw - m), axis=1, keepdims=True)) + m
    out_ref[...] = lw - lse


def _gather_body(tab_hbm, idx_hbm, out_hbm, idx_v, tab_v,
                 o0, o1, o2, o3, o4):
    c = lax.axis_index("c")
    s = lax.axis_index("s")
    wid = s * 2 + c
    b = wid // 2
    h = wid % 2
    pltpu.sync_copy(idx_hbm.at[pl.ds(b * N + h * HALF, HALF)], idx_v)
    pltpu.sync_copy(tab_hbm.at[pl.ds(b * (8 * N), 8 * N)], tab_v)
    outs = (o0, o1, o2, o3, o4)

    def body(j, carry):
        iv = idx_v[pl.ds(j * 16, 16)]
        for r in range(5):
            vals = plsc.load_gather(tab_v, [iv + np.int32(r * N)])
            outs[r][pl.ds(j * 16, 16)] = vals
        return carry

    lax.fori_loop(0, HALF // 16, body, 0)
    for r in range(5):
        pltpu.sync_copy(
            outs[r],
            out_hbm.at[pl.ds((b * 5 + r) * N + h * HALF, HALF)])


def _sc_gather(tab, idx):
    mesh = plsc.VectorSubcoreMesh(core_axis_name="c", subcore_axis_name="s")
    kern = pl.kernel(
        _gather_body,
        out_type=jax.ShapeDtypeStruct((B * 5 * N,), jnp.float32),
        mesh=mesh,
        compiler_params=pltpu.CompilerParams(needs_layout_passes=False),
        scratch_types=[
            pltpu.VMEM((HALF,), jnp.int32),
            pltpu.VMEM((8 * N,), jnp.float32),
        ] + [pltpu.VMEM((HALF,), jnp.float32) for _ in range(5)],
    )
    return kern(tab.reshape(-1), idx.reshape(-1)).reshape(B, 5, N)


@jax.jit
def kernel(x_prev, v_prev, log_w_prev, z_t, anchors):
    xt = jnp.swapaxes(x_prev, 1, 2)
    vt = jnp.swapaxes(v_prev, 1, 2)
    lw3 = log_w_prev[:, None, :]
    z3 = z_t[:, :, None]

    tab, lsq3, xest3 = pl.pallas_call(
        _weights_body,
        grid=(B,),
        in_specs=[
            pl.BlockSpec((1, 3, N), lambda i: (i, 0, 0)),
            pl.BlockSpec((1, 3, N), lambda i: (i, 0, 0)),
            pl.BlockSpec((1, 1, N), lambda i: (i, 0, 0)),
            pl.BlockSpec((1, A, 1), lambda i: (i, 0, 0)),
            pl.BlockSpec((A, 3), lambda i: (0, 0)),
        ],
        out_specs=[
            pl.BlockSpec((1, 8, N), lambda i: (i, 0, 0)),
            pl.BlockSpec((1, 1, N), lambda i: (i, 0, 0)),
            pl.BlockSpec((1, 3, 1), lambda i: (i, 0, 0)),
        ],
        out_shape=[
            jax.ShapeDtypeStruct((B, 8, N), jnp.float32),
            jax.ShapeDtypeStruct((B, 1, N), jnp.float32),
            jax.ShapeDtypeStruct((B, 3, 1), jnp.float32),
        ],
    )(xt, vt, lw3, z3, anchors)

    lsq = lsq3[:, 0, :]

    u_sc = _sc_uniform()

    na_rows = (N - S_SC) * B
    idx_a = pl.pallas_call(
        _argmax_body,
        grid=(na_rows // ROWS,),
        in_specs=[pl.BlockSpec((B, N), lambda g: (0, 0))],
        out_specs=pl.BlockSpec((ROWS, 1), lambda g: (g, 0)),
        out_shape=jax.ShapeDtypeStruct((na_rows, 1), jnp.int32),
    )(lsq)

    idx_b = pl.pallas_call(
        _argmax_u_body,
        grid=(SC_ROWS // ROWS,),
        in_specs=[
            pl.BlockSpec((ROWS, N), lambda g: (g, 0)),
            pl.BlockSpec((B, N), lambda g: (0, 0)),
        ],
        out_specs=pl.BlockSpec((ROWS, 1), lambda g: (g, 0)),
        out_shape=jax.ShapeDtypeStruct((SC_ROWS, 1), jnp.int32),
    )(u_sc, lsq)

    idx_col = jnp.concatenate([idx_a, idx_b], axis=0)
    idx = idx_col.reshape(N, B).T

    gathered = _sc_gather(tab, idx)
    x_res = jnp.swapaxes(gathered[:, 0:3, :], 1, 2)
    w_sel = gathered[:, 3, :]
    q_sel = gathered[:, 4, :]

    log_w_res = pl.pallas_call(
        _final_body,
        in_specs=[
            pl.BlockSpec((B, N), lambda: (0, 0)),
            pl.BlockSpec((B, N), lambda: (0, 0)),
        ],
        out_specs=pl.BlockSpec((B, N), lambda: (0, 0)),
        out_shape=jax.ShapeDtypeStruct((B, N), jnp.float32),
    )(w_sel, q_sel)

    x_est = xest3.reshape(B, 3)
    return (x_res, log_w_res, x_est)
